# baseline (speedup 1.0000x reference)
import dataclasses
import functools

import jax
import jax.numpy as jnp
from jax import lax
from jax.experimental import pallas as pl
from jax.experimental.pallas import tpu as pltpu
from jax.experimental.pallas import tpu_sc as plsc

_PAD = 1
_EPS = 1e-05

_NC = 2
_NS = 16
_NW = _NC * _NS
_CH = 32
_L = 16


def _prologue_body(ids_ref, pos_ref, type_ref, oid_ref, opos_ref):
    ids = ids_ref[...]
    mask = (ids != _PAD).astype(jnp.int32)
    x = mask
    seq = ids.shape[1]
    k = 1
    while k < seq:
        shifted = jnp.concatenate(
            [jnp.zeros((ids.shape[0], k), jnp.int32), x[:, :-k]], axis=1)
        x = x + shifted
        k *= 2
    oid_ref[...] = x * mask + _PAD
    opos_ref[...] = pos_ref[...] + type_ref[0:1, :]


def _prologue(input_ids, pos_emb, type_emb):
    return pl.pallas_call(
        _prologue_body,
        out_shape=[
            jax.ShapeDtypeStruct(input_ids.shape, jnp.int32),
            jax.ShapeDtypeStruct(pos_emb.shape, jnp.float32),
        ],
    )(input_ids, pos_emb, type_emb)


@functools.lru_cache(maxsize=None)
def _make_fused(v_word, v_pos, d, b):
    rpw = b // _NW
    nch = rpw // _CH
    assert nch % 2 == 0
    inv_d = 1.0 / d
    mesh = plsc.VectorSubcoreMesh(core_axis_name="c", subcore_axis_name="s")
    cp = pltpu.CompilerParams()
    if "needs_layout_passes" in pltpu.CompilerParams.__dataclass_fields__:
        cp = dataclasses.replace(cp, needs_layout_passes=False)

    @functools.partial(
        pl.kernel,
        mesh=mesh,
        compiler_params=cp,
        out_type=jax.ShapeDtypeStruct((b, d), jnp.float32),
        scratch_types=[
            pltpu.VMEM((rpw,), jnp.int32),
            pltpu.VMEM((rpw,), jnp.int32),
            pltpu.VMEM((2, _CH, d), jnp.float32),
            pltpu.VMEM((2, _CH, d), jnp.float32),
            pltpu.VMEM((d,), jnp.float32),
            pltpu.VMEM((d,), jnp.float32),
        ] + [pltpu.SemaphoreType.DMA] * 6,
    )
    def fused_kernel(word_hbm, pos_hbm, iw_hbm, ip_hbm, lw_hbm, lb_hbm,
                     out_hbm, iw_v, ip_v, wbuf, pbuf, lw_v, lb_v,
                     sw0, sw1, sp0, sp1, so0, so1):
        semw = (sw0, sw1)
        semp = (sp0, sp1)
        semo = (so0, so1)
        wid = lax.axis_index("s") * _NC + lax.axis_index("c")
        base = wid * rpw
        pltpu.sync_copy(iw_hbm.at[pl.ds(base, rpw)], iw_v)
        pltpu.sync_copy(ip_hbm.at[pl.ds(base, rpw)], ip_v)
        pltpu.sync_copy(lw_hbm, lw_v)
        pltpu.sync_copy(lb_hbm, lb_v)

        def fire(cc, bb):
            pltpu.async_copy(
                word_hbm.at[iw_v.at[pl.ds(cc * _CH, _CH)]], wbuf.at[bb],
                semw[bb])
            pltpu.async_copy(
                pos_hbm.at[ip_v.at[pl.ds(cc * _CH, _CH)]], pbuf.at[bb],
                semp[bb])

        def wait_gather(bb):
            pltpu.make_async_copy(
                word_hbm.at[pl.ds(0, _CH)], wbuf.at[bb], semw[bb]).wait()
            pltpu.make_async_copy(
                pos_hbm.at[pl.ds(0, _CH)], pbuf.at[bb], semp[bb]).wait()

        def wait_out(bb):
            pltpu.make_async_copy(
                wbuf.at[bb], out_hbm.at[pl.ds(base, _CH)], semo[bb]).wait()

        fire(0, 0)

        @pl.loop(0, nch, step=2)
        def _(c):
            for bb in range(2):
                cc = c + bb
                nb = 1 - bb

                @pl.when(cc + 1 < nch)
                def _():
                    @pl.when(cc >= 1)
                    def _():
                        wait_out(nb)

                    fire(cc + 1, nb)

                wait_gather(bb)

                @plsc.parallel_loop(0, _CH, step=1, unroll=2)
                def _(r):
                    s = jnp.zeros((_L,), jnp.float32)
                    s2 = jnp.zeros((_L,), jnp.float32)
                    for col in range(0, d, _L):
                        x = (wbuf[bb, r, pl.ds(col, _L)]
                             + pbuf[bb, r, pl.ds(col, _L)])
                        wbuf[bb, r, pl.ds(col, _L)] = x
                        s = s + x
                        s2 = s2 + x * x
                    mean = jnp.sum(s) * inv_d
                    var = jnp.sum(s2) * inv_d - mean * mean
                    vv = jnp.full((_L,), var + _EPS, jnp.float32)
                    mv = jnp.full((_L,), mean, jnp.float32)
                    y = plsc.bitcast(
                        jnp.int32(0x5F3759DF)
                        - (plsc.bitcast(vv, jnp.int32) >> 1),
                        jnp.float32)
                    for _ in range(3):
                        y = y * (1.5 - 0.5 * vv * y * y)
                    for col in range(0, d, _L):
                        x = wbuf[bb, r, pl.ds(col, _L)]
                        wbuf[bb, r, pl.ds(col, _L)] = (
                            (x - mv) * y * lw_v[pl.ds(col, _L)]
                            + lb_v[pl.ds(col, _L)])

                pltpu.async_copy(
                    wbuf.at[bb], out_hbm.at[pl.ds(base + cc * _CH, _CH)],
                    semo[bb])

        wait_out(0)
        wait_out(1)

    return fused_kernel


def kernel(input_ids, word_emb, pos_emb, type_emb, ln_w, ln_b):
    bb, seq = input_ids.shape
    d = word_emb.shape[1]
    b = bb * seq

    position_ids, pos_plus = _prologue(input_ids, pos_emb, type_emb)
    ids_flat = input_ids.reshape(b)
    pos_flat = position_ids.reshape(b)

    fused = _make_fused(word_emb.shape[0], pos_emb.shape[0], d, b)
    out = fused(word_emb, pos_plus, ids_flat, pos_flat, ln_w, ln_b)
    return out.reshape(bb, seq, d)


# speedup vs baseline: 1.0301x; 1.0301x over previous
import dataclasses
import functools

import jax
import jax.numpy as jnp
from jax import lax
from jax.experimental import pallas as pl
from jax.experimental.pallas import tpu as pltpu
from jax.experimental.pallas import tpu_sc as plsc

_PAD = 1
_EPS = 1e-05

_NC = 2
_NS = 16
_NW = _NC * _NS
_CH = 32
_L = 16


def _lane_shuffle(x, idx):
    return lax.gather(
        x, idx[:, None],
        dimension_numbers=lax.GatherDimensionNumbers(
            offset_dims=(), collapsed_slice_dims=(0,), start_index_map=(0,)),
        slice_sizes=(1,),
        mode=lax.GatherScatterMode.PROMISE_IN_BOUNDS)


def _lane_sum(x):
    for sh in (8, 4, 2, 1):
        idx = jnp.arange(_L, dtype=jnp.int32) ^ sh
        x = x + _lane_shuffle(x, idx)
    return x


def _prologue_body(ids_ref, pos_ref, type_ref, oid_ref, opos_ref):
    ids = ids_ref[...]
    mask = (ids != _PAD).astype(jnp.int32)
    x = mask
    seq = ids.shape[1]
    k = 1
    while k < seq:
        shifted = jnp.concatenate(
            [jnp.zeros((ids.shape[0], k), jnp.int32), x[:, :-k]], axis=1)
        x = x + shifted
        k *= 2
    oid_ref[...] = x * mask + _PAD
    opos_ref[...] = pos_ref[...] + type_ref[0:1, :]


def _prologue(input_ids, pos_emb, type_emb):
    return pl.pallas_call(
        _prologue_body,
        out_shape=[
            jax.ShapeDtypeStruct(input_ids.shape, jnp.int32),
            jax.ShapeDtypeStruct(pos_emb.shape, jnp.float32),
        ],
    )(input_ids, pos_emb, type_emb)


@functools.lru_cache(maxsize=None)
def _make_fused(v_word, v_pos, d, b):
    rpw = b // _NW
    nch = rpw // _CH
    assert nch % 2 == 0
    inv_d = 1.0 / d
    mesh = plsc.VectorSubcoreMesh(core_axis_name="c", subcore_axis_name="s")
    cp = pltpu.CompilerParams()
    if "needs_layout_passes" in pltpu.CompilerParams.__dataclass_fields__:
        cp = dataclasses.replace(cp, needs_layout_passes=False)

    @functools.partial(
        pl.kernel,
        mesh=mesh,
        compiler_params=cp,
        out_type=jax.ShapeDtypeStruct((b, d), jnp.float32),
        scratch_types=[
            pltpu.VMEM((rpw,), jnp.int32),
            pltpu.VMEM((rpw,), jnp.int32),
            pltpu.VMEM((2, _CH, d), jnp.float32),
            pltpu.VMEM((2, _CH, d), jnp.float32),
            pltpu.VMEM((d,), jnp.float32),
            pltpu.VMEM((d,), jnp.float32),
        ] + [pltpu.SemaphoreType.DMA] * 6,
    )
    def fused_kernel(word_hbm, pos_hbm, iw_hbm, ip_hbm, lw_hbm, lb_hbm,
                     out_hbm, iw_v, ip_v, wbuf, pbuf, lw_v, lb_v,
                     sw0, sw1, sp0, sp1, so0, so1):
        semw = (sw0, sw1)
        semp = (sp0, sp1)
        semo = (so0, so1)
        wid = lax.axis_index("s") * _NC + lax.axis_index("c")
        base = wid * rpw
        pltpu.sync_copy(iw_hbm.at[pl.ds(base, rpw)], iw_v)
        pltpu.sync_copy(ip_hbm.at[pl.ds(base, rpw)], ip_v)
        pltpu.sync_copy(lw_hbm, lw_v)
        pltpu.sync_copy(lb_hbm, lb_v)

        def fire(cc, bb):
            pltpu.async_copy(
                word_hbm.at[iw_v.at[pl.ds(cc * _CH, _CH)]], wbuf.at[bb],
                semw[bb])
            pltpu.async_copy(
                pos_hbm.at[ip_v.at[pl.ds(cc * _CH, _CH)]], pbuf.at[bb],
                semp[bb])

        def wait_gather(bb):
            pltpu.make_async_copy(
                word_hbm.at[pl.ds(0, _CH)], wbuf.at[bb], semw[bb]).wait()
            pltpu.make_async_copy(
                pos_hbm.at[pl.ds(0, _CH)], pbuf.at[bb], semp[bb]).wait()

        def wait_out(bb):
            pltpu.make_async_copy(
                wbuf.at[bb], out_hbm.at[pl.ds(base, _CH)], semo[bb]).wait()

        fire(0, 0)

        @pl.loop(0, nch, step=2)
        def _(c):
            for bb in range(2):
                cc = c + bb
                nb = 1 - bb

                @pl.when(cc + 1 < nch)
                def _():
                    @pl.when(cc >= 1)
                    def _():
                        wait_out(nb)

                    fire(cc + 1, nb)

                wait_gather(bb)

                @plsc.parallel_loop(0, _CH, step=1, unroll=2)
                def _(r):
                    s = jnp.zeros((_L,), jnp.float32)
                    s2 = jnp.zeros((_L,), jnp.float32)
                    for col in range(0, d, _L):
                        x = (wbuf[bb, r, pl.ds(col, _L)]
                             + pbuf[bb, r, pl.ds(col, _L)])
                        wbuf[bb, r, pl.ds(col, _L)] = x
                        s = s + x
                        s2 = s2 + x * x
                    mv = _lane_sum(s) * inv_d
                    vv = _lane_sum(s2) * inv_d - mv * mv + _EPS
                    y = plsc.bitcast(
                        jnp.int32(0x5F3759DF)
                        - (plsc.bitcast(vv, jnp.int32) >> 1),
                        jnp.float32)
                    for _ in range(3):
                        y = y * (1.5 - 0.5 * vv * y * y)
                    for col in range(0, d, _L):
                        x = wbuf[bb, r, pl.ds(col, _L)]
                        wbuf[bb, r, pl.ds(col, _L)] = (
                            (x - mv) * y * lw_v[pl.ds(col, _L)]
                            + lb_v[pl.ds(col, _L)])

                pltpu.async_copy(
                    wbuf.at[bb], out_hbm.at[pl.ds(base + cc * _CH, _CH)],
                    semo[bb])

        wait_out(0)
        wait_out(1)

    return fused_kernel


def kernel(input_ids, word_emb, pos_emb, type_emb, ln_w, ln_b):
    bb, seq = input_ids.shape
    d = word_emb.shape[1]
    b = bb * seq

    position_ids, pos_plus = _prologue(input_ids, pos_emb, type_emb)
    ids_flat = input_ids.reshape(b)
    pos_flat = position_ids.reshape(b)

    fused = _make_fused(word_emb.shape[0], pos_emb.shape[0], d, b)
    out = fused(word_emb, pos_plus, ids_flat, pos_flat, ln_w, ln_b)
    return out.reshape(bb, seq, d)


# speedup vs baseline: 1.4618x; 1.4191x over previous
import dataclasses
import functools

import jax
import jax.numpy as jnp
from jax import lax
from jax.experimental import pallas as pl
from jax.experimental.pallas import tpu as pltpu
from jax.experimental.pallas import tpu_sc as plsc

_PAD = 1
_EPS = 1e-05
_HIDDEN = 768

_NC = 2
_NS = 16
_NW = _NC * _NS
_CH = 32


def _posid_body(ids_ref, oid_ref):
    ids = ids_ref[...]
    mask = (ids != _PAD).astype(jnp.int32)
    x = mask
    seq = ids.shape[1]
    k = 1
    while k < seq:
        shifted = jnp.concatenate(
            [jnp.zeros((ids.shape[0], k), jnp.int32), x[:, :-k]], axis=1)
        x = x + shifted
        k *= 2
    oid_ref[...] = x * mask + _PAD


def _position_ids(input_ids):
    return pl.pallas_call(
        _posid_body,
        out_shape=jax.ShapeDtypeStruct(input_ids.shape, jnp.int32),
    )(input_ids)


def _pack_pos_table(pos_emb):
    v, d = pos_emb.shape
    p = pos_emb.astype(jnp.bfloat16)
    p = p.reshape(v, d // 32, 2, 16).transpose(0, 1, 3, 2).reshape(v, d // 2, 2)
    return lax.bitcast_convert_type(p, jnp.int32)


@functools.lru_cache(maxsize=None)
def _make_gather_add(v_word, v_pos, d, b):
    rpw = b // _NW
    nch = rpw // _CH
    assert nch % 2 == 0
    mesh = plsc.VectorSubcoreMesh(core_axis_name="c", subcore_axis_name="s")
    cp = pltpu.CompilerParams()
    if "needs_layout_passes" in pltpu.CompilerParams.__dataclass_fields__:
        cp = dataclasses.replace(cp, needs_layout_passes=False)

    @functools.partial(
        pl.kernel,
        mesh=mesh,
        compiler_params=cp,
        out_type=jax.ShapeDtypeStruct((b, d), jnp.float32),
        scratch_types=[
            pltpu.VMEM((rpw,), jnp.int32),
            pltpu.VMEM((rpw,), jnp.int32),
            pltpu.VMEM((2, _CH, d), jnp.float32),
            pltpu.VMEM((2, _CH, d // 2), jnp.int32),
        ] + [pltpu.SemaphoreType.DMA] * 6,
    )
    def gather_kernel(word_hbm, pos_hbm, iw_hbm, ip_hbm, out_hbm,
                      iw_v, ip_v, wbuf, pbuf,
                      sw0, sw1, sp0, sp1, so0, so1):
        semw = (sw0, sw1)
        semp = (sp0, sp1)
        semo = (so0, so1)
        wid = lax.axis_index("s") * _NC + lax.axis_index("c")
        base = wid * rpw

        def fire(cc, bb):
            pltpu.async_copy(
                word_hbm.at[iw_v.at[pl.ds(cc * _CH, _CH)]], wbuf.at[bb],
                semw[bb])
            pltpu.async_copy(
                pos_hbm.at[ip_v.at[pl.ds(cc * _CH, _CH)]], pbuf.at[bb],
                semp[bb])

        def wait_gather(bb):
            pltpu.make_async_copy(
                word_hbm.at[pl.ds(0, _CH)], wbuf.at[bb], semw[bb]).wait()
            pltpu.make_async_copy(
                pos_hbm.at[pl.ds(0, _CH)], pbuf.at[bb], semp[bb]).wait()

        def wait_out(bb):
            pltpu.make_async_copy(
                wbuf.at[bb], out_hbm.at[pl.ds(base, _CH)], semo[bb]).wait()

        pltpu.sync_copy(iw_hbm.at[pl.ds(base, rpw)], iw_v)
        pltpu.sync_copy(ip_hbm.at[pl.ds(base, rpw)], ip_v)
        fire(0, 0)

        @pl.loop(0, nch, step=2)
        def _(c):
            for bb in range(2):
                cc = c + bb
                nb = 1 - bb

                @pl.when(cc + 1 < nch)
                def _():
                    @pl.when(cc >= 1)
                    def _():
                        wait_out(nb)

                    fire(cc + 1, nb)

                wait_gather(bb)

                @plsc.parallel_loop(0, _CH, step=1, unroll=2)
                def _(r):
                    for col in range(0, d, 32):
                        pa, pb = plsc.unpack(
                            plsc.bitcast(pbuf[bb, r, pl.ds(col // 2, 16)],
                                         jnp.bfloat16),
                            format=plsc.PackFormat.INTERLEAVED,
                            preferred_element_type=jnp.float32)
                        wbuf[bb, r, pl.ds(col, 16)] = (
                            wbuf[bb, r, pl.ds(col, 16)] + pa)
                        wbuf[bb, r, pl.ds(col + 16, 16)] = (
                            wbuf[bb, r, pl.ds(col + 16, 16)] + pb)

                pltpu.async_copy(
                    wbuf.at[bb], out_hbm.at[pl.ds(base + cc * _CH, _CH)],
                    semo[bb])

        wait_out(0)
        wait_out(1)

    return gather_kernel


def _ln_body(s_ref, t_ref, lw_ref, lb_ref, o_ref):
    x = s_ref[...] + t_ref[0:1, :]
    mean = jnp.mean(x, axis=-1, keepdims=True)
    m2 = jnp.mean(x * x, axis=-1, keepdims=True)
    var = m2 - mean * mean
    o_ref[...] = (x - mean) * lax.rsqrt(var + _EPS) * lw_ref[...] + lb_ref[...]


def _ln(sum_rows, type_emb, ln_w, ln_b):
    b, d = sum_rows.shape
    rb = 1024
    grid = (b // rb,)
    return pl.pallas_call(
        _ln_body,
        grid=grid,
        in_specs=[
            pl.BlockSpec((rb, d), lambda i: (i, 0)),
            pl.BlockSpec(type_emb.shape, lambda i: (0, 0)),
            pl.BlockSpec((1, d), lambda i: (0, 0)),
            pl.BlockSpec((1, d), lambda i: (0, 0)),
        ],
        out_specs=pl.BlockSpec((rb, d), lambda i: (i, 0)),
        out_shape=jax.ShapeDtypeStruct((b, d), jnp.float32),
    )(sum_rows, type_emb, ln_w, ln_b)


def kernel(input_ids, word_emb, pos_emb, type_emb, ln_w, ln_b):
    bb, seq = input_ids.shape
    d = word_emb.shape[1]
    b = bb * seq

    position_ids = _position_ids(input_ids)
    pos_packed = _pack_pos_table(pos_emb)
    ids_flat = input_ids.reshape(b)
    pos_flat = position_ids.reshape(b)

    gather = _make_gather_add(word_emb.shape[0], pos_emb.shape[0], d, b)
    sum_rows = gather(word_emb, pos_packed, ids_flat, pos_flat)

    out = _ln(sum_rows, type_emb,
              ln_w.reshape(1, d), ln_b.reshape(1, d))
    return out.reshape(bb, seq, d)


# speedup vs baseline: 1.5744x; 1.0770x over previous
import dataclasses
import functools

import jax
import jax.numpy as jnp
from jax import lax
from jax.experimental import pallas as pl
from jax.experimental.pallas import tpu as pltpu
from jax.experimental.pallas import tpu_sc as plsc

_PAD = 1
_EPS = 1e-05
_HIDDEN = 768

_NC = 2
_NS = 16
_NW = _NC * _NS
_CH = 32


def _posid_body(ids_ref, oid_ref):
    ids = ids_ref[...]
    mask = (ids != _PAD).astype(jnp.int32)
    x = mask
    seq = ids.shape[1]
    k = 1
    while k < seq:
        shifted = jnp.concatenate(
            [jnp.zeros((ids.shape[0], k), jnp.int32), x[:, :-k]], axis=1)
        x = x + shifted
        k *= 2
    oid_ref[...] = x * mask + _PAD


def _position_ids(input_ids):
    return pl.pallas_call(
        _posid_body,
        out_shape=jax.ShapeDtypeStruct(input_ids.shape, jnp.int32),
    )(input_ids)


@functools.lru_cache(maxsize=None)
def _make_gather_add(v_word, v_pos, d, b):
    rpw = b // _NW
    nch = rpw // _CH
    assert nch % 2 == 0
    mesh = plsc.VectorSubcoreMesh(core_axis_name="c", subcore_axis_name="s")
    cp = pltpu.CompilerParams()
    if "needs_layout_passes" in pltpu.CompilerParams.__dataclass_fields__:
        cp = dataclasses.replace(cp, needs_layout_passes=False)

    @functools.partial(
        pl.kernel,
        mesh=mesh,
        compiler_params=cp,
        out_type=jax.ShapeDtypeStruct((b, d), jnp.float32),
        scratch_types=[
            pltpu.VMEM((rpw,), jnp.int32),
            pltpu.VMEM((rpw,), jnp.int32),
            pltpu.VMEM((2, _CH, d), jnp.float32),
            pltpu.VMEM((2, _CH, d), jnp.float32),
        ] + [pltpu.SemaphoreType.DMA] * 6,
    )
    def gather_kernel(word_hbm, pos_hbm, iw_hbm, ip_hbm, out_hbm,
                      iw_v, ip_v, wbuf, pbuf,
                      sw0, sw1, sp0, sp1, so0, so1):
        semw = (sw0, sw1)
        semp = (sp0, sp1)
        semo = (so0, so1)
        wid = lax.axis_index("s") * _NC + lax.axis_index("c")
        base = wid * rpw

        def fire(cc, bb):
            pltpu.async_copy(
                word_hbm.at[iw_v.at[pl.ds(cc * _CH, _CH)]], wbuf.at[bb],
                semw[bb])
            pltpu.async_copy(
                pos_hbm.at[ip_v.at[pl.ds(cc * _CH, _CH)]], pbuf.at[bb],
                semp[bb])

        def wait_gather(bb):
            pltpu.make_async_copy(
                word_hbm.at[pl.ds(0, _CH)], wbuf.at[bb], semw[bb]).wait()
            pltpu.make_async_copy(
                pos_hbm.at[pl.ds(0, _CH)], pbuf.at[bb], semp[bb]).wait()

        def wait_out(bb):
            pltpu.make_async_copy(
                wbuf.at[bb], out_hbm.at[pl.ds(base, _CH)], semo[bb]).wait()

        pltpu.sync_copy(iw_hbm.at[pl.ds(base, rpw)], iw_v)
        pltpu.sync_copy(ip_hbm.at[pl.ds(base, rpw)], ip_v)
        fire(0, 0)

        @pl.loop(0, nch, step=2)
        def _(c):
            for bb in range(2):
                cc = c + bb
                nb = 1 - bb

                @pl.when(cc + 1 < nch)
                def _():
                    @pl.when(cc >= 1)
                    def _():
                        wait_out(nb)

                    fire(cc + 1, nb)

                wait_gather(bb)

                @plsc.parallel_loop(0, _CH, step=1, unroll=2)
                def _(r):
                    for col in range(0, d, 16):
                        wbuf[bb, r, pl.ds(col, 16)] = (
                            wbuf[bb, r, pl.ds(col, 16)]
                            + pbuf[bb, r, pl.ds(col, 16)])

                pltpu.async_copy(
                    wbuf.at[bb], out_hbm.at[pl.ds(base + cc * _CH, _CH)],
                    semo[bb])

        wait_out(0)
        wait_out(1)

    return gather_kernel


def _ln_body(s_ref, t_ref, lw_ref, lb_ref, o_ref):
    x = s_ref[...] + t_ref[0:1, :]
    mean = jnp.mean(x, axis=-1, keepdims=True)
    m2 = jnp.mean(x * x, axis=-1, keepdims=True)
    var = m2 - mean * mean
    o_ref[...] = (x - mean) * lax.rsqrt(var + _EPS) * lw_ref[...] + lb_ref[...]


def _ln(sum_rows, type_emb, ln_w, ln_b):
    b, d = sum_rows.shape
    rb = 2048
    grid = (b // rb,)
    return pl.pallas_call(
        _ln_body,
        grid=grid,
        in_specs=[
            pl.BlockSpec((rb, d), lambda i: (i, 0)),
            pl.BlockSpec(type_emb.shape, lambda i: (0, 0)),
            pl.BlockSpec((1, d), lambda i: (0, 0)),
            pl.BlockSpec((1, d), lambda i: (0, 0)),
        ],
        out_specs=pl.BlockSpec((rb, d), lambda i: (i, 0)),
        out_shape=jax.ShapeDtypeStruct((b, d), jnp.float32),
    )(sum_rows, type_emb, ln_w, ln_b)


def kernel(input_ids, word_emb, pos_emb, type_emb, ln_w, ln_b):
    bb, seq = input_ids.shape
    d = word_emb.shape[1]
    b = bb * seq

    position_ids = _position_ids(input_ids)
    ids_flat = input_ids.reshape(b)
    pos_flat = position_ids.reshape(b)

    gather = _make_gather_add(word_emb.shape[0], pos_emb.shape[0], d, b)
    sum_rows = gather(word_emb, pos_emb, ids_flat, pos_flat)

    out = _ln(sum_rows, type_emb,
              ln_w.reshape(1, d), ln_b.reshape(1, d))
    return out.reshape(bb, seq, d)
